# baseline (speedup 1.0000x reference)
import jax
import jax.numpy as jnp
from jax.experimental import pallas as pl


def _body(x_ref, o_ref):
    x = x_ref[...]
    o_ref[...] = jnp.where(x != 0.0, jnp.float32(2.0), x)


def kernel(x):
    m, n = x.shape
    block_m = 512
    return pl.pallas_call(
        _body,
        grid=(m // block_m,),
        in_specs=[pl.BlockSpec((block_m, n), lambda i: (i, 0))],
        out_specs=pl.BlockSpec((block_m, n), lambda i: (i, 0)),
        out_shape=jax.ShapeDtypeStruct((m, n), x.dtype),
    )(x)
